# baseline (speedup 1.0000x reference)
import jax
import jax.numpy as jnp
from jax import lax
from jax.experimental import pallas as pl
from jax.experimental.pallas import tpu as pltpu
from jax.experimental.pallas import tpu_sc as plsc

BATCH = 16384
N_ROWS = 1000000
EMB = 32
HID = 128
FLAT = N_ROWS * EMB

_info = plsc.get_sparse_core_info()
_NC, _NS = _info.num_cores, _info.num_subcores
_NW = _NC * _NS
_BPW = BATCH // _NW
_CH = 128
_NCH = _BPW * EMB // _CH


def _gather_body(uoff_hbm, boff_hbm, ut_hbm, bt_hbm, u_out, b_out,
                 uoff_v, boff_v, uval_v, bval_v, sem):
    wid = lax.axis_index("s") * _NC + lax.axis_index("c")
    pltpu.sync_copy(uoff_hbm.at[wid], uoff_v)
    pltpu.sync_copy(boff_hbm.at[wid], boff_v)

    @pl.loop(0, _NCH)
    def _fire(j):
        pltpu.async_copy(ut_hbm.at[uoff_v.at[j]], uval_v.at[j], sem)
        pltpu.async_copy(bt_hbm.at[boff_v.at[j]], bval_v.at[j], sem)

    pltpu.make_async_copy(u_out.at[wid], uval_v, sem).wait()
    pltpu.make_async_copy(b_out.at[wid], bval_v, sem).wait()
    pltpu.sync_copy(uval_v, u_out.at[wid])
    pltpu.sync_copy(bval_v, b_out.at[wid])


_gather = pl.kernel(
    _gather_body,
    mesh=plsc.VectorSubcoreMesh(core_axis_name="c", subcore_axis_name="s"),
    out_type=[
        jax.ShapeDtypeStruct((_NW, _NCH, _CH), jnp.float32),
        jax.ShapeDtypeStruct((_NW, _NCH, _CH), jnp.float32),
    ],
    scratch_types=[
        pltpu.VMEM((_NCH, _CH), jnp.int32),
        pltpu.VMEM((_NCH, _CH), jnp.int32),
        pltpu.VMEM((_NCH, _CH), jnp.float32),
        pltpu.VMEM((_NCH, _CH), jnp.float32),
        pltpu.SemaphoreType.DMA,
    ],
)

_BLK = 2048


def _mlp_body(u_ref, bk_ref, w1u_ref, w1b_ref, b1_ref, w2_ref, b2_ref, o_ref):
    h = jnp.dot(u_ref[...], w1u_ref[...], preferred_element_type=jnp.float32)
    h = h + jnp.dot(bk_ref[...], w1b_ref[...], preferred_element_type=jnp.float32)
    h = jnp.maximum(h + b1_ref[...], 0.0)
    o_ref[...] = jnp.sum(h * w2_ref[...], axis=1) + b2_ref[0, 0]


def _mlp(u, bk, w1u, w1b, b1, w2, b2):
    grid = BATCH // _BLK
    return pl.pallas_call(
        _mlp_body,
        grid=(grid,),
        in_specs=[
            pl.BlockSpec((_BLK, EMB), lambda i: (i, 0)),
            pl.BlockSpec((_BLK, EMB), lambda i: (i, 0)),
            pl.BlockSpec((EMB, HID), lambda i: (0, 0)),
            pl.BlockSpec((EMB, HID), lambda i: (0, 0)),
            pl.BlockSpec((1, HID), lambda i: (0, 0)),
            pl.BlockSpec((1, HID), lambda i: (0, 0)),
            pl.BlockSpec(memory_space=pltpu.SMEM),
        ],
        out_specs=pl.BlockSpec((_BLK,), lambda i: (i,)),
        out_shape=jax.ShapeDtypeStruct((BATCH,), jnp.float32),
    )(u, bk, w1u, w1b, b1, w2, b2)


def kernel(user, book, user_table, book_table, W1, b1, W2, b2):
    user = user.astype(jnp.int32)
    book = book.astype(jnp.int32)
    dim_off = (jnp.arange(EMB, dtype=jnp.int32) * N_ROWS)[None, :]
    uoff = (user[:, None] + dim_off).reshape(_NW, _NCH, _CH)
    boff = (book[:, None] + dim_off).reshape(_NW, _NCH, _CH)
    uflat = user_table.T.reshape(FLAT)
    bflat = book_table.T.reshape(FLAT)
    u3, b3 = _gather(uoff, boff, uflat, bflat)
    u = u3.reshape(BATCH, EMB)
    bk = b3.reshape(BATCH, EMB)
    w1t = W1.T
    w1u = w1t[:EMB]
    w1b = w1t[EMB:]
    b1r = b1.reshape(1, HID)
    w2r = W2.reshape(1, HID)
    b2r = b2.reshape(1, 1)
    return _mlp(u, bk, w1u, w1b, b1r, w2r, b2r)


# speedup vs baseline: 5.4920x; 5.4920x over previous
import jax
import jax.numpy as jnp
from jax import lax
from jax.experimental import pallas as pl
from jax.experimental.pallas import tpu as pltpu
from jax.experimental.pallas import tpu_sc as plsc

BATCH = 16384
N_ROWS = 1000000
EMB = 32
HID = 128
ROWW = 128

_info = plsc.get_sparse_core_info()
_NC, _NS = _info.num_cores, _info.num_subcores
_NW = _NC * _NS
_BPW = BATCH // _NW
_CH = 128
_NCH = _BPW // _CH

def _gather_body(uidx_hbm, bidx_hbm, ut_hbm, bt_hbm, u_out, b_out,
                 uidx_v, bidx_v, rows_v, sem):
    wid = lax.axis_index("s") * _NC + lax.axis_index("c")
    base = wid * _BPW
    pltpu.sync_copy(uidx_hbm.at[pl.ds(wid * _NCH, _NCH)], uidx_v)
    pltpu.sync_copy(bidx_hbm.at[pl.ds(wid * _NCH, _NCH)], bidx_v)
    copies = []
    for j in range(_NCH):
        copies.append(pltpu.async_copy(
            ut_hbm.at[uidx_v.at[j]], rows_v.at[pl.ds(j * _CH, _CH)], sem))
    for c in copies:
        c.wait()
    pltpu.sync_copy(rows_v, u_out.at[pl.ds(base, _BPW)])
    copies = []
    for j in range(_NCH):
        copies.append(pltpu.async_copy(
            bt_hbm.at[bidx_v.at[j]], rows_v.at[pl.ds(j * _CH, _CH)], sem))
    for c in copies:
        c.wait()
    pltpu.sync_copy(rows_v, b_out.at[pl.ds(base, _BPW)])


_gather = pl.kernel(
    _gather_body,
    mesh=plsc.VectorSubcoreMesh(core_axis_name="c", subcore_axis_name="s"),
    out_type=[
        jax.ShapeDtypeStruct((BATCH, ROWW), jnp.float32),
        jax.ShapeDtypeStruct((BATCH, ROWW), jnp.float32),
    ],
    scratch_types=[
        pltpu.VMEM((_NCH, _CH), jnp.int32),
        pltpu.VMEM((_NCH, _CH), jnp.int32),
        pltpu.VMEM((_BPW, ROWW), jnp.float32),
        pltpu.SemaphoreType.DMA,
    ],
    compiler_params=pltpu.CompilerParams(use_tc_tiling_on_sc=True),
)

_BLK = 2048


def _pick(q, x):
    return jnp.where(
        q < 2,
        jnp.where(q == 0, x[:, 0:EMB], x[:, EMB:2 * EMB]),
        jnp.where(q == 2, x[:, 2 * EMB:3 * EMB], x[:, 3 * EMB:]),
    )


def _mlp_body(u_ref, bk_ref, qu_ref, qb_ref, w1u_ref, w1b_ref, b1_ref,
              w2_ref, b2_ref, o_ref):
    u = _pick(qu_ref[...], u_ref[...])
    bk = _pick(qb_ref[...], bk_ref[...])
    h = jnp.dot(u, w1u_ref[...], preferred_element_type=jnp.float32)
    h = h + jnp.dot(bk, w1b_ref[...], preferred_element_type=jnp.float32)
    h = jnp.maximum(h + b1_ref[...], 0.0)
    o_ref[...] = jnp.sum(h * w2_ref[...], axis=1) + b2_ref[0, 0]


def _mlp(u, bk, qu, qb, w1u, w1b, b1, w2, b2):
    grid = BATCH // _BLK
    return pl.pallas_call(
        _mlp_body,
        grid=(grid,),
        in_specs=[
            pl.BlockSpec((_BLK, ROWW), lambda i: (i, 0)),
            pl.BlockSpec((_BLK, ROWW), lambda i: (i, 0)),
            pl.BlockSpec((_BLK, 1), lambda i: (i, 0)),
            pl.BlockSpec((_BLK, 1), lambda i: (i, 0)),
            pl.BlockSpec((EMB, HID), lambda i: (0, 0)),
            pl.BlockSpec((EMB, HID), lambda i: (0, 0)),
            pl.BlockSpec((1, HID), lambda i: (0, 0)),
            pl.BlockSpec((1, HID), lambda i: (0, 0)),
            pl.BlockSpec(memory_space=pltpu.SMEM),
        ],
        out_specs=pl.BlockSpec((_BLK,), lambda i: (i,)),
        out_shape=jax.ShapeDtypeStruct((BATCH,), jnp.float32),
    )(u, bk, qu, qb, w1u, w1b, b1, w2, b2)


def kernel(user, book, user_table, book_table, W1, b1, W2, b2):
    user = user.astype(jnp.int32)
    book = book.astype(jnp.int32)
    uidx = (user >> 2).reshape(_NW * _NCH, _CH)
    bidx = (book >> 2).reshape(_NW * _NCH, _CH)
    qu = (user & 3).reshape(BATCH, 1)
    qb = (book & 3).reshape(BATCH, 1)
    utp = user_table.reshape(N_ROWS // 4, ROWW)
    btp = book_table.reshape(N_ROWS // 4, ROWW)
    u, bk = _gather(uidx, bidx, utp, btp)
    w1t = W1.T
    w1u = w1t[:EMB]
    w1b = w1t[EMB:]
    b1r = b1.reshape(1, HID)
    w2r = W2.reshape(1, HID)
    b2r = b2.reshape(1, 1)
    return _mlp(u, bk, qu, qb, w1u, w1b, b1r, w2r, b2r)


# speedup vs baseline: 9.4398x; 1.7188x over previous
import jax
import jax.numpy as jnp
from jax import lax
from jax.experimental import pallas as pl
from jax.experimental.pallas import tpu as pltpu
from jax.experimental.pallas import tpu_sc as plsc

BATCH = 16384
N_ROWS = 1000000
EMB = 32
HID = 128
ROWW = 128

_info = plsc.get_sparse_core_info()
_NC, _NS = _info.num_cores, _info.num_subcores
_NW = _NC * _NS
_BPW = BATCH // _NW
_CH = 128
_NCH = _BPW // _CH

_QSUP = 250880
_TRC = 5120
_TRG = _QSUP // _TRC


def _tr_body(u0, u1, u2, u3, b0, b1, b2, b3, uo_ref, bo_ref):
    for q, (ur, br) in enumerate(zip((u0, u1, u2, u3), (b0, b1, b2, b3))):
        uo_ref[:, q * EMB:(q + 1) * EMB] = ur[...].T
        bo_ref[:, q * EMB:(q + 1) * EMB] = br[...].T


def _transpose(utT, btT):
    in_specs = []
    for q in range(4):
        in_specs.append(pl.BlockSpec((EMB, _TRC), lambda i, q=q: (0, q * _TRG + i)))
    for q in range(4):
        in_specs.append(pl.BlockSpec((EMB, _TRC), lambda i, q=q: (0, q * _TRG + i)))
    return pl.pallas_call(
        _tr_body,
        grid=(_TRG,),
        in_specs=in_specs,
        out_specs=[
            pl.BlockSpec((_TRC, ROWW), lambda i: (i, 0)),
            pl.BlockSpec((_TRC, ROWW), lambda i: (i, 0)),
        ],
        out_shape=[
            jax.ShapeDtypeStruct((_QSUP, ROWW), jnp.float32),
            jax.ShapeDtypeStruct((_QSUP, ROWW), jnp.float32),
        ],
    )(utT, utT, utT, utT, btT, btT, btT, btT)


def _gather_body(uidx_hbm, bidx_hbm, ut_hbm, bt_hbm, u_out, b_out,
                 uidx_v, bidx_v, rows_v, sem):
    wid = lax.axis_index("s") * _NC + lax.axis_index("c")
    base = wid * _BPW
    pltpu.sync_copy(uidx_hbm.at[pl.ds(wid * _NCH, _NCH)], uidx_v)
    pltpu.sync_copy(bidx_hbm.at[pl.ds(wid * _NCH, _NCH)], bidx_v)
    copies = []
    for j in range(_NCH):
        copies.append(pltpu.async_copy(
            ut_hbm.at[uidx_v.at[j]], rows_v.at[pl.ds(j * _CH, _CH)], sem))
    for c in copies:
        c.wait()
    pltpu.sync_copy(rows_v, u_out.at[pl.ds(base, _BPW)])
    copies = []
    for j in range(_NCH):
        copies.append(pltpu.async_copy(
            bt_hbm.at[bidx_v.at[j]], rows_v.at[pl.ds(j * _CH, _CH)], sem))
    for c in copies:
        c.wait()
    pltpu.sync_copy(rows_v, b_out.at[pl.ds(base, _BPW)])


_gather = pl.kernel(
    _gather_body,
    mesh=plsc.VectorSubcoreMesh(core_axis_name="c", subcore_axis_name="s"),
    out_type=[
        jax.ShapeDtypeStruct((BATCH, ROWW), jnp.float32),
        jax.ShapeDtypeStruct((BATCH, ROWW), jnp.float32),
    ],
    scratch_types=[
        pltpu.VMEM((_NCH, _CH), jnp.int32),
        pltpu.VMEM((_NCH, _CH), jnp.int32),
        pltpu.VMEM((_BPW, ROWW), jnp.float32),
        pltpu.SemaphoreType.DMA,
    ],
    compiler_params=pltpu.CompilerParams(use_tc_tiling_on_sc=True),
)

_BLK = 2048


def _pick(q, x):
    return jnp.where(
        q < 2,
        jnp.where(q == 0, x[:, 0:EMB], x[:, EMB:2 * EMB]),
        jnp.where(q == 2, x[:, 2 * EMB:3 * EMB], x[:, 3 * EMB:]),
    )


def _mlp_body(u_ref, bk_ref, qu_ref, qb_ref, w1u_ref, w1b_ref, b1_ref,
              w2_ref, b2_ref, o_ref):
    u = _pick(qu_ref[...], u_ref[...])
    bk = _pick(qb_ref[...], bk_ref[...])
    h = jnp.dot(u, w1u_ref[...], preferred_element_type=jnp.float32)
    h = h + jnp.dot(bk, w1b_ref[...], preferred_element_type=jnp.float32)
    h = jnp.maximum(h + b1_ref[...], 0.0)
    o_ref[...] = jnp.sum(h * w2_ref[...], axis=1) + b2_ref[0, 0]


def _mlp(u, bk, qu, qb, w1u, w1b, b1, w2, b2):
    grid = BATCH // _BLK
    return pl.pallas_call(
        _mlp_body,
        grid=(grid,),
        in_specs=[
            pl.BlockSpec((_BLK, ROWW), lambda i: (i, 0)),
            pl.BlockSpec((_BLK, ROWW), lambda i: (i, 0)),
            pl.BlockSpec((_BLK, 1), lambda i: (i, 0)),
            pl.BlockSpec((_BLK, 1), lambda i: (i, 0)),
            pl.BlockSpec((EMB, HID), lambda i: (0, 0)),
            pl.BlockSpec((EMB, HID), lambda i: (0, 0)),
            pl.BlockSpec((1, HID), lambda i: (0, 0)),
            pl.BlockSpec((1, HID), lambda i: (0, 0)),
            pl.BlockSpec(memory_space=pltpu.SMEM),
        ],
        out_specs=pl.BlockSpec((_BLK,), lambda i: (i,)),
        out_shape=jax.ShapeDtypeStruct((BATCH,), jnp.float32),
    )(u, bk, qu, qb, w1u, w1b, b1, w2, b2)


def kernel(user, book, user_table, book_table, W1, b1, W2, b2):
    user = user.astype(jnp.int32)
    book = book.astype(jnp.int32)
    uidx = (user % _QSUP).reshape(_NW * _NCH, _CH)
    bidx = (book % _QSUP).reshape(_NW * _NCH, _CH)
    qu = (user // _QSUP).reshape(BATCH, 1)
    qb = (book // _QSUP).reshape(BATCH, 1)
    utp, btp = _transpose(user_table.T, book_table.T)
    u, bk = _gather(uidx, bidx, utp, btp)
    w1t = W1.T
    w1u = w1t[:EMB]
    w1b = w1t[EMB:]
    b1r = b1.reshape(1, HID)
    w2r = W2.reshape(1, HID)
    b2r = b2.reshape(1, 1)
    return _mlp(u, bk, qu, qb, w1u, w1b, b1r, w2r, b2r)
